# Initial kernel scaffold; baseline (speedup 1.0000x reference)
import jax
import jax.numpy as jnp
from jax.experimental import pallas as pl


def kernel(x1, edge_index1, x2, edge_index2, W1, b1, W2, b2, Wc1, bc1, Wc2, bc2):
    raise NotImplementedError("write your pallas kernel here")


# speedup vs baseline: 11.6142x; 11.6142x over previous
import functools

import jax
import jax.numpy as jnp
from jax import lax
from jax.experimental import pallas as pl
from jax.experimental.pallas import tpu as pltpu, tpu_sc as plsc

N = 10000
NPAD = 10240
E = 160000
EB = 128
NROWS = 1280
IN_DIM = 256
HIDDEN = 512
FC = 128
NB = 1000
f32 = jnp.float32
i32 = jnp.int32

@functools.lru_cache(maxsize=None)
def _mesh():
    return plsc.VectorSubcoreMesh(core_axis_name="c", subcore_axis_name="s")


def _tile_rows(w, per, extra):
    start = per * w + jnp.minimum(w, extra)
    return start


@functools.lru_cache(maxsize=None)
def _make_deg():
    return functools.partial(
        pl.kernel,
        out_type=(
            jax.ShapeDtypeStruct((2, NPAD, 16), f32),
            jax.ShapeDtypeStruct((2, NPAD, 16), f32),
        ),
        mesh=_mesh(),
        scratch_types=[
            pltpu.VMEM_SHARED((NPAD, 16), f32),
            pltpu.VMEM((40, EB), i32),
            pltpu.VMEM((EB, 16), f32),
            pltpu.VMEM((EB, 16), f32),
        ],
    )(_deg_body)


def _deg_body(dst1, dst2, out1, out2, acc, dstb, ones, zbuf):
    c = lax.axis_index("c")
    s = lax.axis_index("s")
    w = c * 16 + s

    def _fill(i, _):
        ones[i, :] = jnp.ones((16,), f32)
        return 0

    lax.fori_loop(0, EB, _fill, 0)

    def _zfill(i, _):
        zbuf[i, :] = jnp.zeros((16,), f32)
        return 0

    lax.fori_loop(0, EB, _zfill, 0)

    per = NROWS // 32
    rstart = per * w

    for dsthbm, outhbm in ((dst1, out1), (dst2, out2)):
        for t in range(5):
            pltpu.sync_copy(zbuf, acc.at[pl.ds(s * 640 + t * EB, EB)])
        plsc.subcore_barrier()
        pltpu.sync_copy(dsthbm.at[pl.ds(rstart, per)], dstb)

        def _body(j, _):
            pltpu.sync_copy(ones, acc.at[dstb.at[j]], add=True)
            return 0

        lax.fori_loop(0, per, _body, 0)
        plsc.subcore_barrier()
        pltpu.sync_copy(acc.at[pl.ds(s * 640, 640)],
                        outhbm.at[c, pl.ds(s * 640, 640)])
        plsc.subcore_barrier()


@functools.lru_cache(maxsize=None)
def _make_agg(C):
    @functools.partial(
        pl.kernel,
        out_type=tuple(jax.ShapeDtypeStruct((NPAD, FC), f32) for _ in range(C)),
        mesh=_mesh(),
        scratch_types=[
            pltpu.VMEM_SHARED((NPAD, FC), f32),
            pltpu.VMEM((80, EB), i32),
            pltpu.VMEM((80, EB), i32),
            pltpu.VMEM((EB, FC), f32),
            pltpu.VMEM((32, FC), f32),
            pltpu.SemaphoreType.DMA,
        ],
    )
    def _agg(src_hbm, dst_hbm, *rest):
        xs = rest[:C]
        outs = rest[C:2 * C]
        acc, srcb, dstb, gbuf, zbuf, sem = rest[2 * C:]
        c = lax.axis_index("c")
        s = lax.axis_index("s")

        def _zfill(i, _):
            for j in range(FC // 16):
                zbuf[i, pl.ds(16 * j, 16)] = jnp.zeros((16,), f32)
            return 0

        lax.fori_loop(0, 32, _zfill, 0)

        per = NROWS // 16
        rstart = per * s
        pltpu.sync_copy(src_hbm.at[pl.ds(rstart, per)], srcb)
        pltpu.sync_copy(dst_hbm.at[pl.ds(rstart, per)], dstb)

        def _chunk(x_hbm, out_hbm):
            for t in range(20):
                pltpu.sync_copy(zbuf, acc.at[pl.ds(s * 640 + t * 32, 32)])
            plsc.subcore_barrier()

            def _body(j, _):
                pltpu.async_copy(x_hbm.at[srcb.at[j]], gbuf, sem).wait()
                pltpu.sync_copy(gbuf, acc.at[dstb.at[j]], add=True)
                return 0

            lax.fori_loop(0, per, _body, 0)
            plsc.subcore_barrier()
            pltpu.sync_copy(acc.at[pl.ds(s * 640, 640)],
                            out_hbm.at[pl.ds(s * 640, 640)])
            plsc.subcore_barrier()

        half = C // 2
        for cc in range(2):
            @pl.when(c == cc)
            def _():
                for k in range(half):
                    _chunk(xs[cc * half + k], outs[cc * half + k])

    return _agg


def _prep_body(dp1, dp2, x1, x2, dinv1, dinv2, x1c0, x1c1, x2c0, x2c1):
    for dp, x, dinv, c0, c1 in ((dp1, x1, dinv1, x1c0, x1c1),
                                (dp2, x2, dinv2, x2c0, x2c1)):
        deg = dp[0] + dp[1] + 1.0
        dv = lax.rsqrt(deg)
        dinv[...] = dv
        xs = x[...] * dv[:, :1]
        c0[...] = xs[:, :FC]
        c1[...] = xs[:, FC:]


def _prep_call(dp1, dp2, x1, x2):
    bs_dp = pl.BlockSpec((2, NB, 16), lambda i: (0, i, 0))
    bs_x = pl.BlockSpec((NB, IN_DIM), lambda i: (i, 0))
    bs_dv = pl.BlockSpec((NB, 16), lambda i: (i, 0))
    bs_c = pl.BlockSpec((NB, FC), lambda i: (i, 0))
    return pl.pallas_call(
        _prep_body,
        grid=(N // NB,),
        in_specs=[bs_dp, bs_dp, bs_x, bs_x],
        out_specs=[bs_dv, bs_dv, bs_c, bs_c, bs_c, bs_c],
        out_shape=[
            jax.ShapeDtypeStruct((NPAD, 16), f32),
            jax.ShapeDtypeStruct((NPAD, 16), f32),
            jax.ShapeDtypeStruct((NPAD, FC), f32),
            jax.ShapeDtypeStruct((NPAD, FC), f32),
            jax.ShapeDtypeStruct((NPAD, FC), f32),
            jax.ShapeDtypeStruct((NPAD, FC), f32),
        ],
    )(dp1, dp2, x1, x2)


def _h1_body(a0, a1, x, dinv, W, b, h1, t0, t1, t2, t3):
    dv = dinv[:, :1]
    s = jnp.concatenate([a0[...], a1[...]], axis=1) * dv + x[...] * (dv * dv)
    h = jnp.maximum(jnp.dot(s, W[...], preferred_element_type=f32) + b[...],
                    0.0)
    h1[...] = h
    ht = h * dv
    t0[...] = ht[:, :FC]
    t1[...] = ht[:, FC:2 * FC]
    t2[...] = ht[:, 2 * FC:3 * FC]
    t3[...] = ht[:, 3 * FC:]


def _h1_call(a0, a1, x, dinv, W, b):
    bs_c = pl.BlockSpec((NB, FC), lambda i: (i, 0))
    bs_x = pl.BlockSpec((NB, IN_DIM), lambda i: (i, 0))
    bs_dv = pl.BlockSpec((NB, 16), lambda i: (i, 0))
    bs_W = pl.BlockSpec((IN_DIM, HIDDEN), lambda i: (0, 0))
    bs_b = pl.BlockSpec((1, HIDDEN), lambda i: (0, 0))
    bs_h = pl.BlockSpec((NB, HIDDEN), lambda i: (i, 0))
    return pl.pallas_call(
        _h1_body,
        grid=(N // NB,),
        in_specs=[bs_c, bs_c, bs_x, bs_dv, bs_W, bs_b],
        out_specs=[bs_h, bs_c, bs_c, bs_c, bs_c],
        out_shape=[jax.ShapeDtypeStruct((N, HIDDEN), f32)]
        + [jax.ShapeDtypeStruct((NPAD, FC), f32) for _ in range(4)],
    )(a0, a1, x, dinv, W, b)


def _h2_body(a0, a1, a2, a3, h1, dinv, W, b, out):
    dv = dinv[:, :1]
    s = (jnp.concatenate([a0[...], a1[...], a2[...], a3[...]], axis=1) * dv
         + h1[...] * (dv * dv))
    h = jnp.maximum(jnp.dot(s, W[...], preferred_element_type=f32) + b[...],
                    0.0)
    part = jnp.sum(h, axis=0, keepdims=True)

    @pl.when(pl.program_id(0) == 0)
    def _():
        out[...] = jnp.zeros_like(out)

    out[...] += part


def _h2_call(a0, a1, a2, a3, h1, dinv, W, b):
    bs_c = pl.BlockSpec((NB, FC), lambda i: (i, 0))
    bs_h = pl.BlockSpec((NB, HIDDEN), lambda i: (i, 0))
    bs_dv = pl.BlockSpec((NB, 16), lambda i: (i, 0))
    bs_W = pl.BlockSpec((HIDDEN, HIDDEN), lambda i: (0, 0))
    bs_b = pl.BlockSpec((1, HIDDEN), lambda i: (0, 0))
    bs_o = pl.BlockSpec((1, HIDDEN), lambda i: (0, 0))
    return pl.pallas_call(
        _h2_body,
        grid=(N // NB,),
        in_specs=[bs_c, bs_c, bs_c, bs_c, bs_h, bs_dv, bs_W, bs_b],
        out_specs=bs_o,
        out_shape=jax.ShapeDtypeStruct((1, HIDDEN), f32),
    )(a0, a1, a2, a3, h1, dinv, W, b)


def _mlp_body(s1, s2, Wc1, bc1, Wc2, bc2, out):
    comb = jnp.concatenate([s1[...], s2[...]], axis=1) * (1.0 / N)
    h = jnp.maximum(
        jnp.dot(comb, Wc1[...], preferred_element_type=f32) + bc1[...], 0.0)
    z = jnp.dot(h, Wc2[...], preferred_element_type=f32) + bc2[...]
    out[...] = jax.nn.sigmoid(z)


def _mlp_call(s1, s2, Wc1, bc1, Wc2, bc2):
    return pl.pallas_call(
        _mlp_body,
        out_shape=jax.ShapeDtypeStruct((1, 1), f32),
    )(s1, s2, Wc1, bc1, Wc2, bc2)


def kernel(x1, edge_index1, x2, edge_index2, W1, b1, W2, b2,
           Wc1, bc1, Wc2, bc2):
    npad_e = NROWS * EB - E
    pad = N + (jnp.arange(npad_e, dtype=i32) % (NPAD - N))
    def _rows(v):
        return jnp.concatenate([v.astype(i32), pad]).reshape(NROWS, EB)
    src1 = _rows(edge_index1[0])
    dst1 = _rows(edge_index1[1])
    src2 = _rows(edge_index2[0])
    dst2 = _rows(edge_index2[1])
    b1r = b1.reshape(1, HIDDEN)
    b2r = b2.reshape(1, HIDDEN)
    bc1r = bc1.reshape(1, 128)
    bc2r = bc2.reshape(1, 1)

    _agg2 = _make_agg(2)
    _agg4 = _make_agg(4)
    dp1, dp2 = _make_deg()(dst1, dst2)
    dinv1, dinv2, x1c0, x1c1, x2c0, x2c1 = _prep_call(dp1, dp2, x1, x2)

    a1_0, a1_1 = _agg2(src1, dst1, x1c0, x1c1)
    a2_0, a2_1 = _agg2(src2, dst2, x2c0, x2c1)

    h1_1, t10, t11, t12, t13 = _h1_call(a1_0, a1_1, x1, dinv1, W1, b1r)
    h1_2, t20, t21, t22, t23 = _h1_call(a2_0, a2_1, x2, dinv2, W1, b1r)

    g1 = _agg4(src1, dst1, t10, t11, t12, t13)
    g2 = _agg4(src2, dst2, t20, t21, t22, t23)

    s1 = _h2_call(*g1, h1_1, dinv1, W2, b2r)
    s2 = _h2_call(*g2, h1_2, dinv2, W2, b2r)

    return _mlp_call(s1, s2, Wc1, bc1r, Wc2, bc2r)


# speedup vs baseline: 15.0346x; 1.2945x over previous
import functools

import jax
import jax.numpy as jnp
from jax import lax
from jax.experimental import pallas as pl
from jax.experimental.pallas import tpu as pltpu, tpu_sc as plsc

N = 10000
NPAD = 10240
E = 160000
EB = 128
NROWS = 1280
IN_DIM = 256
HIDDEN = 512
FC = 128
NB = 1000
f32 = jnp.float32
i32 = jnp.int32

@functools.lru_cache(maxsize=None)
def _mesh():
    return plsc.VectorSubcoreMesh(core_axis_name="c", subcore_axis_name="s")


def _tile_rows(w, per, extra):
    start = per * w + jnp.minimum(w, extra)
    return start


@functools.lru_cache(maxsize=None)
def _make_deg():
    return functools.partial(
        pl.kernel,
        out_type=(
            jax.ShapeDtypeStruct((2, NPAD, 16), f32),
            jax.ShapeDtypeStruct((2, NPAD, 16), f32),
        ),
        mesh=_mesh(),
        scratch_types=[
            pltpu.VMEM_SHARED((NPAD, 16), f32),
            pltpu.VMEM((40, EB), i32),
            pltpu.VMEM((EB, 16), f32),
            pltpu.VMEM((EB, 16), f32),
        ],
    )(_deg_body)


def _deg_body(dst1, dst2, out1, out2, acc, dstb, ones, zbuf):
    c = lax.axis_index("c")
    s = lax.axis_index("s")
    w = c * 16 + s

    def _fill(i, _):
        ones[i, :] = jnp.ones((16,), f32)
        return 0

    lax.fori_loop(0, EB, _fill, 0)

    def _zfill(i, _):
        zbuf[i, :] = jnp.zeros((16,), f32)
        return 0

    lax.fori_loop(0, EB, _zfill, 0)

    per = NROWS // 32
    rstart = per * w

    for dsthbm, outhbm in ((dst1, out1), (dst2, out2)):
        for t in range(5):
            pltpu.sync_copy(zbuf, acc.at[pl.ds(s * 640 + t * EB, EB)])
        plsc.subcore_barrier()
        pltpu.sync_copy(dsthbm.at[pl.ds(rstart, per)], dstb)

        def _body(j, _):
            pltpu.sync_copy(ones, acc.at[dstb.at[j]], add=True)
            return 0

        lax.fori_loop(0, per, _body, 0)
        plsc.subcore_barrier()
        pltpu.sync_copy(acc.at[pl.ds(s * 640, 640)],
                        outhbm.at[c, pl.ds(s * 640, 640)])
        plsc.subcore_barrier()


@functools.lru_cache(maxsize=None)
def _make_agg(C):
    @functools.partial(
        pl.kernel,
        out_type=tuple(jax.ShapeDtypeStruct((NPAD, FC), f32) for _ in range(C)),
        mesh=_mesh(),
        scratch_types=[
            pltpu.VMEM_SHARED((NPAD, FC), f32),
            pltpu.VMEM((40, EB), i32),
            pltpu.VMEM((40, EB), i32),
            pltpu.VMEM((EB, FC), f32),
            pltpu.VMEM((EB, FC), f32),
            pltpu.VMEM((32, FC), f32),
            pltpu.SemaphoreType.DMA,
            pltpu.SemaphoreType.DMA,
        ],
    )
    def _agg(src_hbm, dst_hbm, *rest):
        xs = rest[:C]
        outs = rest[C:2 * C]
        acc, srcb, dstb, gb0, gb1, zbuf, sem0, sem1 = rest[2 * C:]
        c = lax.axis_index("c")
        s = lax.axis_index("s")

        def _zfill(i, _):
            for j in range(FC // 16):
                zbuf[i, pl.ds(16 * j, 16)] = jnp.zeros((16,), f32)
            return 0

        lax.fori_loop(0, 32, _zfill, 0)

        per = NROWS // 16
        hrows = per // 2

        def _chunk(x_hbm, out_hbm):
            for t in range(20):
                pltpu.sync_copy(zbuf, acc.at[pl.ds(s * 640 + t * 32, 32)])
            plsc.subcore_barrier()

            def _start(j, gb, sem):
                pltpu.async_copy(x_hbm.at[srcb.at[j]], gb, sem)

            def _wait(j, gb, sem):
                pltpu.make_async_copy(x_hbm.at[srcb.at[j]], gb, sem).wait()

            for h in range(2):
                rstart = per * s + hrows * h
                pltpu.sync_copy(src_hbm.at[pl.ds(rstart, hrows)], srcb)
                pltpu.sync_copy(dst_hbm.at[pl.ds(rstart, hrows)], dstb)
                _start(0, gb0, sem0)

                def _body(jj, _):
                    j = 2 * jj
                    _wait(j, gb0, sem0)
                    _start(j + 1, gb1, sem1)
                    pltpu.sync_copy(gb0, acc.at[dstb.at[j]], add=True)
                    _wait(j + 1, gb1, sem1)

                    @pl.when(jj < hrows // 2 - 1)
                    def _():
                        _start(j + 2, gb0, sem0)

                    pltpu.sync_copy(gb1, acc.at[dstb.at[j + 1]], add=True)
                    return 0

                lax.fori_loop(0, hrows // 2, _body, 0)
            plsc.subcore_barrier()
            pltpu.sync_copy(acc.at[pl.ds(s * 640, 640)],
                            out_hbm.at[pl.ds(s * 640, 640)])
            plsc.subcore_barrier()

        half = C // 2
        for cc in range(2):
            @pl.when(c == cc)
            def _():
                for k in range(half):
                    _chunk(xs[cc * half + k], outs[cc * half + k])

    return _agg


def _prep_body(dp1, dp2, x1, x2, dinv1, dinv2, x1c0, x1c1, x2c0, x2c1):
    for dp, x, dinv, c0, c1 in ((dp1, x1, dinv1, x1c0, x1c1),
                                (dp2, x2, dinv2, x2c0, x2c1)):
        deg = dp[0] + dp[1] + 1.0
        dv = lax.rsqrt(deg)
        dinv[...] = dv
        xs = x[...] * dv[:, :1]
        c0[...] = xs[:, :FC]
        c1[...] = xs[:, FC:]


def _prep_call(dp1, dp2, x1, x2):
    bs_dp = pl.BlockSpec((2, NB, 16), lambda i: (0, i, 0))
    bs_x = pl.BlockSpec((NB, IN_DIM), lambda i: (i, 0))
    bs_dv = pl.BlockSpec((NB, 16), lambda i: (i, 0))
    bs_c = pl.BlockSpec((NB, FC), lambda i: (i, 0))
    return pl.pallas_call(
        _prep_body,
        grid=(N // NB,),
        in_specs=[bs_dp, bs_dp, bs_x, bs_x],
        out_specs=[bs_dv, bs_dv, bs_c, bs_c, bs_c, bs_c],
        out_shape=[
            jax.ShapeDtypeStruct((NPAD, 16), f32),
            jax.ShapeDtypeStruct((NPAD, 16), f32),
            jax.ShapeDtypeStruct((NPAD, FC), f32),
            jax.ShapeDtypeStruct((NPAD, FC), f32),
            jax.ShapeDtypeStruct((NPAD, FC), f32),
            jax.ShapeDtypeStruct((NPAD, FC), f32),
        ],
    )(dp1, dp2, x1, x2)


def _h1_body(a0, a1, x, dinv, W, b, h1, t0, t1, t2, t3):
    dv = dinv[:, :1]
    s = jnp.concatenate([a0[...], a1[...]], axis=1) * dv + x[...] * (dv * dv)
    h = jnp.maximum(jnp.dot(s, W[...], preferred_element_type=f32) + b[...],
                    0.0)
    h1[...] = h
    ht = h * dv
    t0[...] = ht[:, :FC]
    t1[...] = ht[:, FC:2 * FC]
    t2[...] = ht[:, 2 * FC:3 * FC]
    t3[...] = ht[:, 3 * FC:]


def _h1_call(a0, a1, x, dinv, W, b):
    bs_c = pl.BlockSpec((NB, FC), lambda i: (i, 0))
    bs_x = pl.BlockSpec((NB, IN_DIM), lambda i: (i, 0))
    bs_dv = pl.BlockSpec((NB, 16), lambda i: (i, 0))
    bs_W = pl.BlockSpec((IN_DIM, HIDDEN), lambda i: (0, 0))
    bs_b = pl.BlockSpec((1, HIDDEN), lambda i: (0, 0))
    bs_h = pl.BlockSpec((NB, HIDDEN), lambda i: (i, 0))
    return pl.pallas_call(
        _h1_body,
        grid=(N // NB,),
        in_specs=[bs_c, bs_c, bs_x, bs_dv, bs_W, bs_b],
        out_specs=[bs_h, bs_c, bs_c, bs_c, bs_c],
        out_shape=[jax.ShapeDtypeStruct((N, HIDDEN), f32)]
        + [jax.ShapeDtypeStruct((NPAD, FC), f32) for _ in range(4)],
    )(a0, a1, x, dinv, W, b)


def _h2_body(a0, a1, a2, a3, h1, dinv, W, b, out):
    dv = dinv[:, :1]
    s = (jnp.concatenate([a0[...], a1[...], a2[...], a3[...]], axis=1) * dv
         + h1[...] * (dv * dv))
    h = jnp.maximum(jnp.dot(s, W[...], preferred_element_type=f32) + b[...],
                    0.0)
    part = jnp.sum(h, axis=0, keepdims=True)

    @pl.when(pl.program_id(0) == 0)
    def _():
        out[...] = jnp.zeros_like(out)

    out[...] += part


def _h2_call(a0, a1, a2, a3, h1, dinv, W, b):
    bs_c = pl.BlockSpec((NB, FC), lambda i: (i, 0))
    bs_h = pl.BlockSpec((NB, HIDDEN), lambda i: (i, 0))
    bs_dv = pl.BlockSpec((NB, 16), lambda i: (i, 0))
    bs_W = pl.BlockSpec((HIDDEN, HIDDEN), lambda i: (0, 0))
    bs_b = pl.BlockSpec((1, HIDDEN), lambda i: (0, 0))
    bs_o = pl.BlockSpec((1, HIDDEN), lambda i: (0, 0))
    return pl.pallas_call(
        _h2_body,
        grid=(N // NB,),
        in_specs=[bs_c, bs_c, bs_c, bs_c, bs_h, bs_dv, bs_W, bs_b],
        out_specs=bs_o,
        out_shape=jax.ShapeDtypeStruct((1, HIDDEN), f32),
    )(a0, a1, a2, a3, h1, dinv, W, b)


def _mlp_body(s1, s2, Wc1, bc1, Wc2, bc2, out):
    comb = jnp.concatenate([s1[...], s2[...]], axis=1) * (1.0 / N)
    h = jnp.maximum(
        jnp.dot(comb, Wc1[...], preferred_element_type=f32) + bc1[...], 0.0)
    z = jnp.dot(h, Wc2[...], preferred_element_type=f32) + bc2[...]
    out[...] = jax.nn.sigmoid(z)


def _mlp_call(s1, s2, Wc1, bc1, Wc2, bc2):
    return pl.pallas_call(
        _mlp_body,
        out_shape=jax.ShapeDtypeStruct((1, 1), f32),
    )(s1, s2, Wc1, bc1, Wc2, bc2)


def kernel(x1, edge_index1, x2, edge_index2, W1, b1, W2, b2,
           Wc1, bc1, Wc2, bc2):
    npad_e = NROWS * EB - E
    pad = N + (jnp.arange(npad_e, dtype=i32) % (NPAD - N))
    def _rows(v):
        return jnp.concatenate([v.astype(i32), pad]).reshape(NROWS, EB)
    src1 = _rows(edge_index1[0])
    dst1 = _rows(edge_index1[1])
    src2 = _rows(edge_index2[0])
    dst2 = _rows(edge_index2[1])
    b1r = b1.reshape(1, HIDDEN)
    b2r = b2.reshape(1, HIDDEN)
    bc1r = bc1.reshape(1, 128)
    bc2r = bc2.reshape(1, 1)

    _agg2 = _make_agg(2)
    _agg4 = _make_agg(4)
    dp1, dp2 = _make_deg()(dst1, dst2)
    dinv1, dinv2, x1c0, x1c1, x2c0, x2c1 = _prep_call(dp1, dp2, x1, x2)

    a1_0, a1_1 = _agg2(src1, dst1, x1c0, x1c1)
    a2_0, a2_1 = _agg2(src2, dst2, x2c0, x2c1)

    h1_1, t10, t11, t12, t13 = _h1_call(a1_0, a1_1, x1, dinv1, W1, b1r)
    h1_2, t20, t21, t22, t23 = _h1_call(a2_0, a2_1, x2, dinv2, W1, b1r)

    g1 = _agg4(src1, dst1, t10, t11, t12, t13)
    g2 = _agg4(src2, dst2, t20, t21, t22, t23)

    s1 = _h2_call(*g1, h1_1, dinv1, W2, b2r)
    s2 = _h2_call(*g2, h1_2, dinv2, W2, b2r)

    return _mlp_call(s1, s2, Wc1, bc1r, Wc2, bc2r)
